# baseline (speedup 1.0000x reference)
import jax
import jax.numpy as jnp
from jax.experimental import pallas as pl
from jax.experimental.pallas import tpu as pltpu

B = 16
T = 4096
S = 128
NA = 4
RB = 16384
NK = T * B // RB
SL = RB // 128
TPB = RB // B


def _body(len_ref, s_ref, a_ref, w_ref, bias_ref, out_ref, acc1, acc2):
    k = pl.program_id(0)

    @pl.when(k == 0)
    def _init():
        acc1[...] = jnp.zeros_like(acc1)
        acc2[...] = jnp.zeros_like(acc2)

    x = s_ref[...]
    lt = jax.lax.dot_general(
        w_ref[...], x, (((1,), (1,)), ((), ())),
        preferred_element_type=jnp.float32,
    )
    rows1 = jax.lax.broadcasted_iota(jnp.int32, (NA, 1), 0)
    bvec = jnp.full((NA, 1), bias_ref[0])
    for a in range(1, NA):
        bvec = jnp.where(rows1 == a, bias_ref[a], bvec)
    lt = lt + bvec
    lt3 = lt.reshape(NA, SL, 128)
    m = jnp.max(lt3, axis=0, keepdims=True)
    ssum = jnp.sum(jnp.exp(lt3 - m), axis=0, keepdims=True)
    lse = m + jnp.log(ssum)

    r_idx = (
        jax.lax.broadcasted_iota(jnp.int32, (1, SL, 128), 1) * 128
        + jax.lax.broadcasted_iota(jnp.int32, (1, SL, 128), 2)
    )
    b_col = jax.lax.bitwise_and(r_idx, B - 1)
    t_col = k * TPB + jax.lax.shift_right_logical(r_idx, 4)
    lenv = jnp.full((1, SL, 128), len_ref[0])
    for b in range(1, B):
        lenv = jnp.where(b_col == b, len_ref[b], lenv)
    mask = t_col < lenv

    acts = a_ref[...]
    arows = jax.lax.broadcasted_iota(jnp.int32, (NA, SL, 128), 0)
    sel = jnp.logical_and(arows == acts, mask)
    acc1[...] += jnp.where(mask, lse, 0.0)
    acc2[...] += jnp.where(sel, lt3, 0.0)

    @pl.when(k == NK - 1)
    def _final():
        out_ref[0, 0] = jnp.sum(acc1[...]) - jnp.sum(acc2[...])


@jax.jit
def kernel(s_i_batch, actions_batch, lengths, W, bias):
    s_flat = jnp.transpose(s_i_batch, (1, 0, 2)).reshape((T + 1) * B, S)
    acts_ti = actions_batch.T.reshape(NK, SL, 128)
    w_t = W.T

    grid_spec = pltpu.PrefetchScalarGridSpec(
        num_scalar_prefetch=1,
        grid=(NK,),
        in_specs=[
            pl.BlockSpec((RB, S), lambda k, L: (k, 0)),
            pl.BlockSpec((1, SL, 128), lambda k, L: (k, 0, 0)),
            pl.BlockSpec((NA, S), lambda k, L: (0, 0)),
            pl.BlockSpec(memory_space=pltpu.SMEM),
        ],
        out_specs=pl.BlockSpec(
            (1, 1), lambda k, L: (0, 0), memory_space=pltpu.SMEM
        ),
        scratch_shapes=[
            pltpu.VMEM((1, SL, 128), jnp.float32),
            pltpu.VMEM((NA, SL, 128), jnp.float32),
        ],
    )
    out = pl.pallas_call(
        _body,
        grid_spec=grid_spec,
        out_shape=jax.ShapeDtypeStruct((1, 1), jnp.float32),
    )(lengths, s_flat, acts_ti, w_t, bias)
    return out[0, 0]


# speedup vs baseline: 1.0200x; 1.0200x over previous
import jax
import jax.numpy as jnp
from jax.experimental import pallas as pl
from jax.experimental.pallas import tpu as pltpu

B = 16
T = 4096
S = 128
NA = 4
RB = 16384
NK = T * B // RB
SL = RB // 128
TPB = RB // B


def _body(len_ref, s_ref, a_ref, w_ref, bias_ref, out_ref, acc1, acc2):
    k = pl.program_id(0)

    @pl.when(k == 0)
    def _init():
        acc1[...] = jnp.zeros_like(acc1)
        acc2[...] = jnp.zeros_like(acc2)

    x = s_ref[...]
    lt = jax.lax.dot_general(
        w_ref[...], x, (((1,), (1,)), ((), ())),
        preferred_element_type=jnp.float32,
    )
    rows1 = jax.lax.broadcasted_iota(jnp.int32, (NA, 1), 0)
    bvec = jnp.full((NA, 1), bias_ref[0])
    for a in range(1, NA):
        bvec = jnp.where(rows1 == a, bias_ref[a], bvec)
    lt = lt + bvec
    lt3 = lt.reshape(NA, SL, 128)
    m = jnp.max(lt3, axis=0, keepdims=True)
    ssum = jnp.sum(jnp.exp(lt3 - m), axis=0, keepdims=True)
    lse = m + jnp.log(ssum)

    r_idx = (
        jax.lax.broadcasted_iota(jnp.int32, (1, SL, 128), 1) * 128
        + jax.lax.broadcasted_iota(jnp.int32, (1, SL, 128), 2)
    )
    b_col = jax.lax.bitwise_and(r_idx, B - 1)
    t_col = k * TPB + jax.lax.shift_right_logical(r_idx, 4)
    lenv = jnp.full((1, SL, 128), len_ref[0])
    for b in range(1, B):
        lenv = jnp.where(b_col == b, len_ref[b], lenv)
    mask = t_col < lenv

    acts = a_ref[...]
    arows = jax.lax.broadcasted_iota(jnp.int32, (NA, SL, 128), 0)
    sel = jnp.logical_and(arows == acts, mask)
    acc1[...] += jnp.where(mask, lse, 0.0)
    acc2[...] += jnp.where(sel, lt3, 0.0)

    @pl.when(k == NK - 1)
    def _final():
        out_ref[0, 0] = jnp.sum(acc1[...]) - jnp.sum(acc2[...])


@jax.jit
def kernel(s_i_batch, actions_batch, lengths, W, bias):
    s_flat = jnp.transpose(s_i_batch, (1, 0, 2)).reshape((T + 1) * B, S)
    acts_ti = (
        actions_batch.reshape(B, NK, TPB)
        .transpose(1, 2, 0)
        .reshape(NK, SL, 128)
    )
    w_t = W.T

    grid_spec = pltpu.PrefetchScalarGridSpec(
        num_scalar_prefetch=1,
        grid=(NK,),
        in_specs=[
            pl.BlockSpec((RB, S), lambda k, L: (k, 0)),
            pl.BlockSpec((1, SL, 128), lambda k, L: (k, 0, 0)),
            pl.BlockSpec((NA, S), lambda k, L: (0, 0)),
            pl.BlockSpec(memory_space=pltpu.SMEM),
        ],
        out_specs=pl.BlockSpec(
            (1, 1), lambda k, L: (0, 0), memory_space=pltpu.SMEM
        ),
        scratch_shapes=[
            pltpu.VMEM((1, SL, 128), jnp.float32),
            pltpu.VMEM((NA, SL, 128), jnp.float32),
        ],
    )
    out = pl.pallas_call(
        _body,
        grid_spec=grid_spec,
        out_shape=jax.ShapeDtypeStruct((1, 1), jnp.float32),
    )(lengths, s_flat, acts_ti, w_t, bias)
    return out[0, 0]
